# Initial kernel scaffold; baseline (speedup 1.0000x reference)
import jax
import jax.numpy as jnp
from jax.experimental import pallas as pl


def kernel(x, node_attrs, edge_index, atomic_numbers, covalent_radii):
    raise NotImplementedError("write your pallas kernel here")


# speedup vs baseline: 426.0197x; 426.0197x over previous
import functools

import jax
import jax.numpy as jnp
from jax import lax
from jax.experimental import pallas as pl
from jax.experimental.pallas import tpu as pltpu
from jax.experimental.pallas import tpu_sc as plsc

Q_EXP = 0.9183
P_EXP = 4.5791
A_COEF = 1.0805

N_NODES = 100000
NELEM = 10
E_EDGES = 6400000

NC = 2
NS = 16
L = 16
NW = NC * NS
EPW = E_EDGES // NW
CHUNK = 4000
NCHUNK = EPW // CHUNK

NODE_BLK = 4000
EW_ROWS = 50000
EW_BLK = 1000


def _node_radii_body(attrs_ref, rad_ref, out_ref):
    attrs = attrs_ref[...]
    best = attrs[:, 0:1]
    r = jnp.full((NODE_BLK, 1), rad_ref[0], dtype=jnp.float32)
    for j in range(1, NELEM):
        c = attrs[:, j:j + 1]
        m = c > best
        best = jnp.where(m, c, best)
        r = jnp.where(m, rad_ref[j], r)
    out_ref[...] = r


def _node_radii(node_attrs, rad_half):
    return pl.pallas_call(
        _node_radii_body,
        grid=(N_NODES // NODE_BLK,),
        in_specs=[
            pl.BlockSpec((NODE_BLK, NELEM), lambda i: (i, 0)),
            pl.BlockSpec(memory_space=pltpu.SMEM),
        ],
        out_specs=pl.BlockSpec((NODE_BLK, 1), lambda i: (i, 0)),
        out_shape=jax.ShapeDtypeStruct((N_NODES, 1), jnp.float32),
    )(node_attrs, rad_half)


@functools.partial(
    pl.kernel,
    mesh=plsc.VectorSubcoreMesh(core_axis_name="c", subcore_axis_name="s"),
    out_type=jax.ShapeDtypeStruct((E_EDGES,), jnp.float32),
    compiler_params=pltpu.CompilerParams(needs_layout_passes=False),
    scratch_types=[
        pltpu.VMEM((N_NODES,), jnp.float32),
        pltpu.VMEM((CHUNK,), jnp.int32),
        pltpu.VMEM((CHUNK,), jnp.int32),
        pltpu.VMEM((CHUNK,), jnp.float32),
    ],
)
def _edge_gather(send_hbm, recv_hbm, rhalf_hbm, out_hbm, tbl, sbuf, rbuf, obuf):
    wid = lax.axis_index("s") * NC + lax.axis_index("c")
    base = wid * EPW
    pltpu.sync_copy(rhalf_hbm, tbl)

    @pl.loop(0, NCHUNK)
    def _chunks(g):
        off = base + g * CHUNK
        pltpu.sync_copy(send_hbm.at[pl.ds(off, CHUNK)], sbuf)
        pltpu.sync_copy(recv_hbm.at[pl.ds(off, CHUNK)], rbuf)

        @pl.loop(0, CHUNK, step=L)
        def _vecs(i):
            s = sbuf[pl.ds(i, L)]
            r = rbuf[pl.ds(i, L)]
            a = plsc.load_gather(tbl, [s])
            b = plsc.load_gather(tbl, [r])
            obuf[pl.ds(i, L)] = a + b

        pltpu.sync_copy(obuf, out_hbm.at[pl.ds(off, CHUNK)])


def _agnesi_body(x_ref, r0_ref, o_ref):
    t = x_ref[...] / r0_ref[...]
    lt = jnp.log(t)
    pq = jnp.exp(Q_EXP * lt)
    pqp = jnp.exp((Q_EXP - P_EXP) * lt)
    o_ref[...] = 1.0 / (1.0 + A_COEF * pq / (1.0 + pqp))


def _agnesi(x2d, r02d):
    return pl.pallas_call(
        _agnesi_body,
        grid=(EW_ROWS // EW_BLK,),
        in_specs=[
            pl.BlockSpec((EW_BLK, 128), lambda i: (i, 0)),
            pl.BlockSpec((EW_BLK, 128), lambda i: (i, 0)),
        ],
        out_specs=pl.BlockSpec((EW_BLK, 128), lambda i: (i, 0)),
        out_shape=jax.ShapeDtypeStruct((EW_ROWS, 128), jnp.float32),
    )(x2d, r02d)


def kernel(x, node_attrs, edge_index, atomic_numbers, covalent_radii):
    edge_index = edge_index.astype(jnp.int32)
    rad_half = (0.5 * covalent_radii)[atomic_numbers.astype(jnp.int32)]
    r_node = _node_radii(node_attrs, rad_half)
    r0 = _edge_gather(edge_index[0], edge_index[1],
                      r_node.reshape(-1))
    y = _agnesi(x.reshape(EW_ROWS, 128), r0.reshape(EW_ROWS, 128))
    return y.reshape(E_EDGES, 1)


# speedup vs baseline: 1251.0043x; 2.9365x over previous
import functools

import jax
import jax.numpy as jnp
from jax import lax
from jax.experimental import pallas as pl
from jax.experimental.pallas import tpu as pltpu
from jax.experimental.pallas import tpu_sc as plsc

Q_EXP = 0.9183
P_EXP = 4.5791
A_COEF = 1.0805

N_NODES = 100000
N_PAD = 102400
NELEM = 10
E_EDGES = 6400000

NC = 2
NS = 16
L = 16
NW = NC * NS
EPW = E_EDGES // NW
CHUNK = 4000
NCHUNK = EPW // CHUNK

NODE_ROWS = 800
NODE_BLK = 80
EW_ROWS = 50000
EW_BLK = 1000


def _node_radii_body(attrs_ref, rad_ref, out_ref):
    best = attrs_ref[0]
    r = jnp.full((NODE_BLK, 128), rad_ref[0], dtype=jnp.float32)
    for j in range(1, NELEM):
        c = attrs_ref[j]
        m = c > best
        best = jnp.where(m, c, best)
        r = jnp.where(m, rad_ref[j], r)
    out_ref[...] = r


def _node_radii(attrs_t, rad_half):
    return pl.pallas_call(
        _node_radii_body,
        grid=(NODE_ROWS // NODE_BLK,),
        in_specs=[
            pl.BlockSpec((NELEM, NODE_BLK, 128), lambda i: (0, i, 0)),
            pl.BlockSpec(memory_space=pltpu.SMEM),
        ],
        out_specs=pl.BlockSpec((NODE_BLK, 128), lambda i: (i, 0)),
        out_shape=jax.ShapeDtypeStruct((NODE_ROWS, 128), jnp.float32),
    )(attrs_t, rad_half)


@functools.partial(
    pl.kernel,
    mesh=plsc.VectorSubcoreMesh(core_axis_name="c", subcore_axis_name="s"),
    out_type=jax.ShapeDtypeStruct((E_EDGES,), jnp.float32),
    compiler_params=pltpu.CompilerParams(needs_layout_passes=False),
    scratch_types=[
        pltpu.VMEM((N_NODES,), jnp.float32),
        pltpu.VMEM((CHUNK,), jnp.int32),
        pltpu.VMEM((CHUNK,), jnp.int32),
        pltpu.VMEM((CHUNK,), jnp.int32),
        pltpu.VMEM((CHUNK,), jnp.int32),
        pltpu.VMEM((CHUNK,), jnp.float32),
        pltpu.VMEM((CHUNK,), jnp.float32),
        pltpu.SemaphoreType.DMA,
        pltpu.SemaphoreType.DMA,
        pltpu.SemaphoreType.DMA,
        pltpu.SemaphoreType.DMA,
])
def _edge_gather(send_hbm, recv_hbm, rhalf_hbm, out_hbm, tbl, sbuf0, sbuf1,
                 rbuf0, rbuf1, obuf0, obuf1, si0, si1, so0, so1):
    wid = lax.axis_index("s") * NC + lax.axis_index("c")
    base = wid * EPW
    sbuf = (sbuf0, sbuf1)
    rbuf = (rbuf0, rbuf1)
    obuf = (obuf0, obuf1)
    si = (si0, si1)
    so = (so0, so1)

    def in_copies(b, g):
        off = base + g * CHUNK
        return (
            pltpu.make_async_copy(send_hbm.at[pl.ds(off, CHUNK)],
                                  sbuf[b], si[b]),
            pltpu.make_async_copy(recv_hbm.at[pl.ds(off, CHUNK)],
                                  rbuf[b], si[b]),
        )

    def out_copy(b, g):
        off = base + g * CHUNK
        return pltpu.make_async_copy(obuf[b],
                                     out_hbm.at[pl.ds(off, CHUNK)], so[b])

    for b in range(2):
        for c in in_copies(b, b):
            c.start()
    pltpu.sync_copy(rhalf_hbm.at[pl.ds(0, N_NODES)], tbl)

    @pl.loop(0, NCHUNK, step=2)
    def _chunks(g0):
        for b in range(2):
            g = g0 + b
            for c in in_copies(b, g):
                c.wait()

            @pl.when(g >= 2)
            def _():
                out_copy(b, g - 2).wait()

            ob = obuf[b]
            sb = sbuf[b]
            rb = rbuf[b]

            @plsc.parallel_loop(0, CHUNK, step=L, unroll=10)
            def _vecs(i):
                s = sb[pl.ds(i, L)]
                r = rb[pl.ds(i, L)]
                ob[pl.ds(i, L)] = (plsc.load_gather(tbl, [s])
                                   + plsc.load_gather(tbl, [r]))

            out_copy(b, g).start()

            @pl.when(g + 2 < NCHUNK)
            def _():
                for c in in_copies(b, g + 2):
                    c.start()

    for b in range(2):
        out_copy(b, NCHUNK - 2 + b).wait()


def _agnesi_body(x_ref, r0_ref, o_ref):
    t = x_ref[...] / r0_ref[...]
    lt = jnp.log(t)
    s = jnp.exp(Q_EXP * lt)
    w = jnp.exp((Q_EXP - P_EXP) * lt)
    o_ref[...] = (1.0 + w) / (1.0 + w + A_COEF * s)


def _agnesi(x2d, r02d):
    return pl.pallas_call(
        _agnesi_body,
        grid=(EW_ROWS // EW_BLK,),
        in_specs=[
            pl.BlockSpec((EW_BLK, 128), lambda i: (i, 0)),
            pl.BlockSpec((EW_BLK, 128), lambda i: (i, 0)),
        ],
        out_specs=pl.BlockSpec((EW_BLK, 128), lambda i: (i, 0)),
        out_shape=jax.ShapeDtypeStruct((EW_ROWS, 128), jnp.float32),
    )(x2d, r02d)


def kernel(x, node_attrs, edge_index, atomic_numbers, covalent_radii):
    edge_index = edge_index.astype(jnp.int32)
    rad_half = (0.5 * covalent_radii)[atomic_numbers.astype(jnp.int32)]
    attrs_t = jnp.pad(node_attrs.T, ((0, 0), (0, N_PAD - N_NODES)))
    attrs_t = attrs_t.reshape(NELEM, NODE_ROWS, 128)
    r_node = _node_radii(attrs_t, rad_half)
    r0 = _edge_gather(edge_index[0], edge_index[1],
                      r_node.reshape(-1))
    y = _agnesi(x.reshape(EW_ROWS, 128), r0.reshape(EW_ROWS, 128))
    return y.reshape(E_EDGES, 1)


# speedup vs baseline: 1395.6352x; 1.1156x over previous
import functools

import jax
import jax.numpy as jnp
from jax import lax
from jax.experimental import pallas as pl
from jax.experimental.pallas import tpu as pltpu
from jax.experimental.pallas import tpu_sc as plsc

Q_EXP = 0.9183
P_EXP = 4.5791
A_COEF = 1.0805

N_NODES = 100000
N_PAD = 102400
NELEM = 10
E_EDGES = 6400000

NC = 2
NS = 16
L = 16
NW = NC * NS
EPW = E_EDGES // NW
CHUNK = 4000
NCHUNK = EPW // CHUNK

NODE_ROWS = 800
NODE_BLK = 80
EW_ROWS = 50000
EW_BLK = 1000


def _node_radii_body(attrs_ref, rad_ref, out_ref):
    best = attrs_ref[0]
    r = jnp.full((NODE_BLK, 128), rad_ref[0], dtype=jnp.float32)
    for j in range(1, NELEM):
        c = attrs_ref[j]
        m = c > best
        best = jnp.where(m, c, best)
        r = jnp.where(m, rad_ref[j], r)
    out_ref[...] = r


def _node_radii(attrs_t, rad_half):
    return pl.pallas_call(
        _node_radii_body,
        grid=(NODE_ROWS // NODE_BLK,),
        in_specs=[
            pl.BlockSpec((NELEM, NODE_BLK, 128), lambda i: (0, i, 0)),
            pl.BlockSpec(memory_space=pltpu.SMEM),
        ],
        out_specs=pl.BlockSpec((NODE_BLK, 128), lambda i: (i, 0)),
        out_shape=jax.ShapeDtypeStruct((NODE_ROWS, 128), jnp.float32),
    )(attrs_t, rad_half)


@functools.partial(
    pl.kernel,
    mesh=plsc.VectorSubcoreMesh(core_axis_name="c", subcore_axis_name="s"),
    out_type=jax.ShapeDtypeStruct((E_EDGES,), jnp.float32),
    compiler_params=pltpu.CompilerParams(needs_layout_passes=False),
    scratch_types=[
        pltpu.VMEM((N_NODES,), jnp.float32),
        pltpu.VMEM((CHUNK,), jnp.int32),
        pltpu.VMEM((CHUNK,), jnp.int32),
        pltpu.VMEM((CHUNK,), jnp.int32),
        pltpu.VMEM((CHUNK,), jnp.int32),
        pltpu.VMEM((CHUNK,), jnp.float32),
        pltpu.VMEM((CHUNK,), jnp.float32),
        pltpu.SemaphoreType.DMA,
        pltpu.SemaphoreType.DMA,
        pltpu.SemaphoreType.DMA,
        pltpu.SemaphoreType.DMA,
])
def _edge_gather(edge_hbm, rhalf_hbm, out_hbm, tbl, sbuf0, sbuf1,
                 rbuf0, rbuf1, obuf0, obuf1, si0, si1, so0, so1):
    wid = lax.axis_index("s") * NC + lax.axis_index("c")
    base = wid * EPW
    sbuf = (sbuf0, sbuf1)
    rbuf = (rbuf0, rbuf1)
    obuf = (obuf0, obuf1)
    si = (si0, si1)
    so = (so0, so1)

    def in_copies(b, g):
        off = base + g * CHUNK
        return (
            pltpu.make_async_copy(edge_hbm.at[pl.ds(off, CHUNK)],
                                  sbuf[b], si[b]),
            pltpu.make_async_copy(edge_hbm.at[pl.ds(E_EDGES + off, CHUNK)],
                                  rbuf[b], si[b]),
        )

    def out_copy(b, g):
        off = base + g * CHUNK
        return pltpu.make_async_copy(obuf[b],
                                     out_hbm.at[pl.ds(off, CHUNK)], so[b])

    for b in range(2):
        for c in in_copies(b, b):
            c.start()
    pltpu.sync_copy(rhalf_hbm.at[pl.ds(0, N_NODES)], tbl)

    @pl.loop(0, NCHUNK, step=2)
    def _chunks(g0):
        for b in range(2):
            g = g0 + b
            for c in in_copies(b, g):
                c.wait()

            @pl.when(g >= 2)
            def _():
                out_copy(b, g - 2).wait()

            ob = obuf[b]
            sb = sbuf[b]
            rb = rbuf[b]

            @plsc.parallel_loop(0, CHUNK, step=L, unroll=10)
            def _vecs(i):
                s = sb[pl.ds(i, L)]
                r = rb[pl.ds(i, L)]
                ob[pl.ds(i, L)] = (plsc.load_gather(tbl, [s])
                                   + plsc.load_gather(tbl, [r]))

            out_copy(b, g).start()

            @pl.when(g + 2 < NCHUNK)
            def _():
                for c in in_copies(b, g + 2):
                    c.start()

    for b in range(2):
        out_copy(b, NCHUNK - 2 + b).wait()


def _agnesi_body(x_ref, r0_ref, o_ref):
    t = x_ref[...] / r0_ref[...]
    lt = jnp.log(t)
    s = jnp.exp(Q_EXP * lt)
    w = jnp.exp((Q_EXP - P_EXP) * lt)
    o_ref[...] = (1.0 + w) / (1.0 + w + A_COEF * s)


def _agnesi(x2d, r02d):
    return pl.pallas_call(
        _agnesi_body,
        grid=(EW_ROWS // EW_BLK,),
        in_specs=[
            pl.BlockSpec((EW_BLK, 128), lambda i: (i, 0)),
            pl.BlockSpec((EW_BLK, 128), lambda i: (i, 0)),
        ],
        out_specs=pl.BlockSpec((EW_BLK, 128), lambda i: (i, 0)),
        out_shape=jax.ShapeDtypeStruct((EW_ROWS, 128), jnp.float32),
    )(x2d, r02d)


def kernel(x, node_attrs, edge_index, atomic_numbers, covalent_radii):
    edge_index = edge_index.astype(jnp.int32)
    rad_half = (0.5 * covalent_radii)[atomic_numbers.astype(jnp.int32)]
    attrs_t = jnp.pad(node_attrs.T, ((0, 0), (0, N_PAD - N_NODES)))
    attrs_t = attrs_t.reshape(NELEM, NODE_ROWS, 128)
    r_node = _node_radii(attrs_t, rad_half)
    r0 = _edge_gather(edge_index.reshape(-1),
                      r_node.reshape(-1))
    y = _agnesi(x.reshape(EW_ROWS, 128), r0.reshape(EW_ROWS, 128))
    return y.reshape(E_EDGES, 1)


# speedup vs baseline: 1400.0978x; 1.0032x over previous
import functools

import jax
import jax.numpy as jnp
from jax import lax
from jax.experimental import pallas as pl
from jax.experimental.pallas import tpu as pltpu
from jax.experimental.pallas import tpu_sc as plsc

Q_EXP = 0.9183
P_EXP = 4.5791
A_COEF = 1.0805

N_NODES = 100000
N_PAD = 102400
NELEM = 10
E_EDGES = 6400000

NC = 2
NS = 16
L = 16
NW = NC * NS
EPW = E_EDGES // NW
CHUNK = 4000
NCHUNK = EPW // CHUNK

NODE_ROWS = 800
NODE_BLK = 80
EW_BLK = 128000


def _node_radii_body(attrs_ref, rad_ref, out_ref):
    best = attrs_ref[0]
    r = jnp.full((NODE_BLK, 128), rad_ref[0], dtype=jnp.float32)
    for j in range(1, NELEM):
        c = attrs_ref[j]
        m = c > best
        best = jnp.where(m, c, best)
        r = jnp.where(m, rad_ref[j], r)
    out_ref[...] = r


def _node_radii(attrs_t, rad_half):
    return pl.pallas_call(
        _node_radii_body,
        grid=(NODE_ROWS // NODE_BLK,),
        in_specs=[
            pl.BlockSpec((NELEM, NODE_BLK, 128), lambda i: (0, i, 0)),
            pl.BlockSpec(memory_space=pltpu.SMEM),
        ],
        out_specs=pl.BlockSpec((NODE_BLK, 128), lambda i: (i, 0)),
        out_shape=jax.ShapeDtypeStruct((NODE_ROWS, 128), jnp.float32),
    )(attrs_t, rad_half)


@functools.partial(
    pl.kernel,
    mesh=plsc.VectorSubcoreMesh(core_axis_name="c", subcore_axis_name="s"),
    out_type=jax.ShapeDtypeStruct((E_EDGES,), jnp.float32),
    compiler_params=pltpu.CompilerParams(needs_layout_passes=False),
    scratch_types=[
        pltpu.VMEM((N_NODES,), jnp.float32),
        pltpu.VMEM((CHUNK,), jnp.int32),
        pltpu.VMEM((CHUNK,), jnp.int32),
        pltpu.VMEM((CHUNK,), jnp.int32),
        pltpu.VMEM((CHUNK,), jnp.int32),
        pltpu.VMEM((CHUNK,), jnp.float32),
        pltpu.VMEM((CHUNK,), jnp.float32),
        pltpu.SemaphoreType.DMA,
        pltpu.SemaphoreType.DMA,
        pltpu.SemaphoreType.DMA,
        pltpu.SemaphoreType.DMA,
])
def _edge_gather(edge_hbm, rhalf_hbm, out_hbm, tbl, sbuf0, sbuf1,
                 rbuf0, rbuf1, obuf0, obuf1, si0, si1, so0, so1):
    wid = lax.axis_index("s") * NC + lax.axis_index("c")
    base = wid * EPW
    sbuf = (sbuf0, sbuf1)
    rbuf = (rbuf0, rbuf1)
    obuf = (obuf0, obuf1)
    si = (si0, si1)
    so = (so0, so1)

    def in_copies(b, g):
        off = base + g * CHUNK
        return (
            pltpu.make_async_copy(edge_hbm.at[pl.ds(off, CHUNK)],
                                  sbuf[b], si[b]),
            pltpu.make_async_copy(edge_hbm.at[pl.ds(E_EDGES + off, CHUNK)],
                                  rbuf[b], si[b]),
        )

    def out_copy(b, g):
        off = base + g * CHUNK
        return pltpu.make_async_copy(obuf[b],
                                     out_hbm.at[pl.ds(off, CHUNK)], so[b])

    for b in range(2):
        for c in in_copies(b, b):
            c.start()
    pltpu.sync_copy(rhalf_hbm.at[pl.ds(0, N_NODES)], tbl)

    @pl.loop(0, NCHUNK, step=2)
    def _chunks(g0):
        for b in range(2):
            g = g0 + b
            for c in in_copies(b, g):
                c.wait()

            @pl.when(g >= 2)
            def _():
                out_copy(b, g - 2).wait()

            ob = obuf[b]
            sb = sbuf[b]
            rb = rbuf[b]

            @plsc.parallel_loop(0, CHUNK, step=L, unroll=10)
            def _vecs(i):
                s = sb[pl.ds(i, L)]
                r = rb[pl.ds(i, L)]
                ob[pl.ds(i, L)] = (plsc.load_gather(tbl, [s])
                                   + plsc.load_gather(tbl, [r]))

            out_copy(b, g).start()

            @pl.when(g + 2 < NCHUNK)
            def _():
                for c in in_copies(b, g + 2):
                    c.start()

    for b in range(2):
        out_copy(b, NCHUNK - 2 + b).wait()


def _agnesi_body(x_ref, r0_ref, o_ref):
    t = x_ref[...] / r0_ref[...]
    lt = jnp.log(t)
    s = jnp.exp(Q_EXP * lt)
    w = jnp.exp((Q_EXP - P_EXP) * lt)
    o_ref[...] = (1.0 + w) / (1.0 + w + A_COEF * s)


def _agnesi(x1d, r01d):
    return pl.pallas_call(
        _agnesi_body,
        grid=(E_EDGES // EW_BLK,),
        in_specs=[
            pl.BlockSpec((EW_BLK,), lambda i: (i,)),
            pl.BlockSpec((EW_BLK,), lambda i: (i,)),
        ],
        out_specs=pl.BlockSpec((EW_BLK,), lambda i: (i,)),
        out_shape=jax.ShapeDtypeStruct((E_EDGES,), jnp.float32),
    )(x1d, r01d)


def kernel(x, node_attrs, edge_index, atomic_numbers, covalent_radii):
    edge_index = edge_index.astype(jnp.int32)
    rad_half = (0.5 * covalent_radii)[atomic_numbers.astype(jnp.int32)]
    attrs_t = jnp.pad(node_attrs.T, ((0, 0), (0, N_PAD - N_NODES)))
    attrs_t = attrs_t.reshape(NELEM, NODE_ROWS, 128)
    r_node = _node_radii(attrs_t, rad_half)
    r0 = _edge_gather(edge_index.reshape(-1),
                      r_node.reshape(-1))
    y = _agnesi(x.reshape(-1), r0)
    return y.reshape(E_EDGES, 1)


# speedup vs baseline: 1581.6849x; 1.1297x over previous
import functools

import jax
import jax.numpy as jnp
from jax import lax
from jax.experimental import pallas as pl
from jax.experimental.pallas import tpu as pltpu
from jax.experimental.pallas import tpu_sc as plsc

Q_EXP = 0.9183
P_EXP = 4.5791
A_COEF = 1.0805

N_NODES = 100000
N_PAD = 102400
NELEM = 10
E_EDGES = 6400000

NC = 2
NS = 16
L = 16
NW = NC * NS
CHUNK = 2560
TCHUNK = E_EDGES // CHUNK
KMAX = (TCHUNK + NW - 1) // NW

NODE_ROWS = 800
NODE_BLK = 80
EW_BLK = 128000


def _node_radii_body(attrs_ref, rad_ref, out_ref):
    best = attrs_ref[0]
    r = jnp.full((NODE_BLK, 128), rad_ref[0], dtype=jnp.float32)
    for j in range(1, NELEM):
        c = attrs_ref[j]
        m = c > best
        best = jnp.where(m, c, best)
        r = jnp.where(m, rad_ref[j], r)
    out_ref[...] = r


def _node_radii(attrs_t, rad_half):
    return pl.pallas_call(
        _node_radii_body,
        grid=(NODE_ROWS // NODE_BLK,),
        in_specs=[
            pl.BlockSpec((NELEM, NODE_BLK, 128), lambda i: (0, i, 0)),
            pl.BlockSpec(memory_space=pltpu.SMEM),
        ],
        out_specs=pl.BlockSpec((NODE_BLK, 128), lambda i: (i, 0)),
        out_shape=jax.ShapeDtypeStruct((NODE_ROWS, 128), jnp.float32),
    )(attrs_t, rad_half)


@functools.partial(
    pl.kernel,
    mesh=plsc.VectorSubcoreMesh(core_axis_name="c", subcore_axis_name="s"),
    out_type=jax.ShapeDtypeStruct((E_EDGES,), jnp.float32),
    compiler_params=pltpu.CompilerParams(needs_layout_passes=False),
    scratch_types=[
        pltpu.VMEM((N_NODES,), jnp.float32),
        pltpu.VMEM((2, CHUNK), jnp.int32),
        pltpu.VMEM((2, CHUNK), jnp.int32),
        pltpu.VMEM((CHUNK,), jnp.float32),
        pltpu.VMEM((CHUNK,), jnp.float32),
        pltpu.SemaphoreType.DMA,
        pltpu.SemaphoreType.DMA,
        pltpu.SemaphoreType.DMA,
        pltpu.SemaphoreType.DMA,
])
def _edge_gather(edge_hbm, rhalf_hbm, out_hbm, tbl, ebuf0, ebuf1,
                 obuf0, obuf1, si0, si1, so0, so1):
    wid = lax.axis_index("s") * NC + lax.axis_index("c")
    ebuf = (ebuf0, ebuf1)
    obuf = (obuf0, obuf1)
    si = (si0, si1)
    so = (so0, so1)

    def valid(k):
        return wid + NW * k < TCHUNK

    def in_copy(b, k):
        off = (wid + NW * k) * CHUNK
        return pltpu.make_async_copy(edge_hbm.at[:, pl.ds(off, CHUNK)],
                                     ebuf[b], si[b])

    def out_copy(b, k):
        off = (wid + NW * k) * CHUNK
        return pltpu.make_async_copy(obuf[b],
                                     out_hbm.at[pl.ds(off, CHUNK)], so[b])

    for b in range(2):
        @pl.when(valid(b))
        def _():
            in_copy(b, b).start()
    pltpu.sync_copy(rhalf_hbm.at[pl.ds(0, N_NODES)], tbl)

    @pl.loop(0, KMAX, step=2)
    def _chunks(k0):
        for b in range(2):
            k = k0 + b

            @pl.when(valid(k))
            def _():
                in_copy(b, k).wait()

                @pl.when(k >= 2)
                def _():
                    out_copy(b, k - 2).wait()

                ob = obuf[b]
                eb = ebuf[b]

                @plsc.parallel_loop(0, CHUNK, step=L, unroll=10)
                def _vecs(i):
                    s = eb[0, pl.ds(i, L)]
                    r = eb[1, pl.ds(i, L)]
                    ob[pl.ds(i, L)] = (plsc.load_gather(tbl, [s])
                                       + plsc.load_gather(tbl, [r]))

                out_copy(b, k).start()

                @pl.when(valid(k + 2))
                def _():
                    in_copy(b, k + 2).start()

    nvalid = (TCHUNK - wid + NW - 1) // NW
    for b in range(2):
        for klast in (nvalid - 1, nvalid - 2):
            @pl.when((klast >= 0) & (klast % 2 == b))
            def _():
                out_copy(b, klast).wait()


def _agnesi_body(x_ref, r0_ref, o_ref):
    t = x_ref[...] / r0_ref[...]
    lt = jnp.log(t)
    s = jnp.exp(Q_EXP * lt)
    w = jnp.exp((Q_EXP - P_EXP) * lt)
    o_ref[...] = (1.0 + w) / (1.0 + w + A_COEF * s)


def _agnesi(x1d, r01d):
    return pl.pallas_call(
        _agnesi_body,
        grid=(E_EDGES // EW_BLK,),
        in_specs=[
            pl.BlockSpec((EW_BLK,), lambda i: (i,)),
            pl.BlockSpec((EW_BLK,), lambda i: (i,)),
        ],
        out_specs=pl.BlockSpec((EW_BLK,), lambda i: (i,)),
        out_shape=jax.ShapeDtypeStruct((E_EDGES,), jnp.float32),
    )(x1d, r01d)


def kernel(x, node_attrs, edge_index, atomic_numbers, covalent_radii):
    edge_index = edge_index.astype(jnp.int32)
    rad_half = (0.5 * covalent_radii)[atomic_numbers.astype(jnp.int32)]
    attrs_t = jnp.pad(node_attrs.T, ((0, 0), (0, N_PAD - N_NODES)))
    attrs_t = attrs_t.reshape(NELEM, NODE_ROWS, 128)
    r_node = _node_radii(attrs_t, rad_half)
    r0 = _edge_gather(edge_index,
                      r_node.reshape(-1))
    y = _agnesi(x.reshape(-1), r0)
    return y.reshape(E_EDGES, 1)
